# baseline (speedup 1.0000x reference)
import functools

import jax
import jax.numpy as jnp
from jax import lax
from jax.experimental import pallas as pl
from jax.experimental.pallas import tpu as pltpu
from jax.experimental.pallas import tpu_sc as plsc

_L = 16


@functools.lru_cache(maxsize=None)
def _build_sc_call(B, N, A, C, AA, SC_ATOMS, interpret=False):
    BB = A - SC_ATOMS
    R = B * N
    F = A * C
    try:
        info = plsc.get_sparse_core_info()
        NC, NS = info.num_cores, info.num_subcores
    except ValueError:
        NC, NS = 2, 16
    mesh = plsc.VectorSubcoreMesh(
        core_axis_name="c", subcore_axis_name="s", num_cores=NC, num_subcores=NS
    )
    NW = NC * NS
    assert R % (NW * _L) == 0
    RES_W = R // NW
    GROUPS = RES_W // _L

    @functools.partial(
        pl.kernel,
        out_type=jax.ShapeDtypeStruct((R * F,), jnp.float32),
        mesh=mesh,
        scratch_types=[
            pltpu.VMEM((RES_W * F,), jnp.float32),
            pltpu.VMEM((RES_W,), jnp.int32),
            pltpu.VMEM((AA * SC_ATOMS,), jnp.int32),
        ],
        compiler_params=pltpu.CompilerParams(needs_layout_passes=False),
        interpret=interpret,
    )
    def sc_call(x_hbm, s_hbm, tbl_hbm, out_hbm, xv, sv, tv):
        wid = lax.axis_index("s") * NC + lax.axis_index("c")
        rbase = wid * RES_W
        fbase = rbase * F
        pltpu.sync_copy(tbl_hbm, tv)
        pltpu.sync_copy(s_hbm.at[pl.ds(rbase, RES_W)], sv)
        pltpu.sync_copy(x_hbm.at[pl.ds(fbase, RES_W * F)], xv)
        lanes = lax.iota(jnp.int32, _L)

        def group(g, carry):
            svec = plsc.load_gather(sv, [g * _L + lanes])
            base = (g * _L + lanes) * F + BB * C
            tidx = svec * SC_ATOMS
            vals = []
            for a in range(SC_ATOMS):
                alt = plsc.load_gather(tv, [tidx + a])
                src = base + alt * C
                for c in range(C):
                    vals.append(plsc.load_gather(xv, [src + c]))
            for a in range(SC_ATOMS):
                for c in range(C):
                    plsc.store_scatter(xv, [base + a * C + c], vals[a * C + c])
            return carry

        lax.fori_loop(0, GROUPS, group, 0)
        pltpu.sync_copy(xv, out_hbm.at[pl.ds(fbase, RES_W * F)])

    return sc_call


def kernel(X, S, symmetry_indices):
    B, N, A, C = X.shape
    AA, SC_ATOMS = symmetry_indices.shape
    x_flat = X.reshape(-1)
    s_flat = S.reshape(-1)
    tbl_flat = symmetry_indices.reshape(-1)
    sc_call = _build_sc_call(B, N, A, C, AA, SC_ATOMS)
    out_flat = sc_call(x_flat, s_flat, tbl_flat)
    return out_flat.reshape(B, N, A, C)


# speedup vs baseline: 92.7432x; 92.7432x over previous
import functools

import jax
import jax.numpy as jnp
from jax import lax
from jax.experimental import pallas as pl
from jax.experimental.pallas import tpu as pltpu
from jax.experimental.pallas import tpu_sc as plsc

_L = 16


@functools.lru_cache(maxsize=None)
def _build_sc_call(B, N, A, C, AA, SC_ATOMS):
    BB = A - SC_ATOMS
    P = A * C
    try:
        info = plsc.get_sparse_core_info()
        NC, NS = info.num_cores, info.num_subcores
    except ValueError:
        NC, NS = 2, 16
    mesh = plsc.VectorSubcoreMesh(
        core_axis_name="c", subcore_axis_name="s", num_cores=NC, num_subcores=NS
    )
    NW = NC * NS
    ROWS = 8
    COLS = B * N // (NW * ROWS)
    assert (B * N) % (NW * ROWS) == 0 and COLS % _L == 0
    WB = B // ROWS
    GROUPS = ROWS * COLS // _L

    @functools.partial(
        pl.kernel,
        out_type=jax.ShapeDtypeStruct((P, B, N), jnp.float32),
        mesh=mesh,
        scratch_types=[
            pltpu.VMEM((P, ROWS, COLS), jnp.float32),
            pltpu.VMEM((ROWS, COLS), jnp.int32),
            pltpu.VMEM((AA * SC_ATOMS,), jnp.int32),
        ],
        compiler_params=pltpu.CompilerParams(needs_layout_passes=False),
    )
    def sc_call(x_hbm, s_hbm, tbl_hbm, out_hbm, xv, sv, tv):
        wid = lax.axis_index("s") * NC + lax.axis_index("c")
        r0 = (wid % WB) * ROWS
        c0 = (wid // WB) * COLS
        pltpu.sync_copy(tbl_hbm, tv)
        pltpu.sync_copy(s_hbm.at[pl.ds(r0, ROWS), pl.ds(c0, COLS)], sv)
        pltpu.sync_copy(x_hbm.at[:, pl.ds(r0, ROWS), pl.ds(c0, COLS)], xv)
        lanes = lax.iota(jnp.int32, _L)

        def group(g, carry):
            s = g // (COLS // _L)
            l0 = (g % (COLS // _L)) * _L
            svec = sv[s, pl.ds(l0, _L)]
            tidx = svec * SC_ATOMS
            svec_s = jnp.full((_L,), s, jnp.int32)
            lvec = l0 + lanes
            vals = []
            for a in range(SC_ATOMS):
                alt = plsc.load_gather(tv, [tidx + a])
                pvec = BB * C + alt * C
                for c in range(C):
                    vals.append(plsc.load_gather(xv, [pvec + c, svec_s, lvec]))
            for a in range(SC_ATOMS):
                for c in range(C):
                    xv[BB * C + a * C + c, s, pl.ds(l0, _L)] = vals[a * C + c]
            return carry

        lax.fori_loop(0, GROUPS, group, 0)
        pltpu.sync_copy(xv, out_hbm.at[:, pl.ds(r0, ROWS), pl.ds(c0, COLS)])

    return sc_call


def kernel(X, S, symmetry_indices):
    B, N, A, C = X.shape
    AA, SC_ATOMS = symmetry_indices.shape
    x_planes = jnp.transpose(X, (2, 3, 0, 1)).reshape(A * C, B, N)
    tbl_flat = symmetry_indices.reshape(-1)
    sc_call = _build_sc_call(B, N, A, C, AA, SC_ATOMS)
    out_planes = sc_call(x_planes, S, tbl_flat)
    return jnp.transpose(out_planes.reshape(A, C, B, N), (2, 3, 0, 1))
